# Initial kernel scaffold; baseline (speedup 1.0000x reference)
import jax
import jax.numpy as jnp
from jax.experimental import pallas as pl


def kernel(logits, targets):
    raise NotImplementedError("write your pallas kernel here")


# speedup vs baseline: 27.7828x; 27.7828x over previous
import functools

import jax
import jax.numpy as jnp
from jax.experimental import pallas as pl
from jax.experimental.pallas import tpu as pltpu

_BIG = 1e9


def _hausdorff_body(logits_ref, targets_ref, out_ref, gt_ref):
    B, H, W = logits_ref.shape
    maskf = (targets_ref[...] != 0).astype(jnp.float32)

    ys = jax.lax.broadcasted_iota(jnp.int32, (1, H, 1), 1).astype(jnp.float32)

    def pass1(ty, g):
        mrow = targets_ref[:, pl.ds(ty, 1), :] != 0
        dy2 = (ys - ty.astype(jnp.float32)) ** 2
        cand = jnp.where(mrow, dy2, _BIG)
        return jnp.minimum(g, cand)

    g0 = jnp.full((B, H, W), _BIG, dtype=jnp.float32)
    g = jax.lax.fori_loop(0, H, pass1, g0)

    gt_ref[...] = jnp.swapaxes(g, 1, 2)

    xs = jax.lax.broadcasted_iota(jnp.int32, (1, W, 1), 1).astype(jnp.float32)

    def pass2(tx, d2t):
        grow = gt_ref[:, pl.ds(tx, 1), :]
        dx2 = (xs - tx.astype(jnp.float32)) ** 2
        cand = grow + dx2
        return jnp.minimum(d2t, cand)

    d20 = jnp.full((B, W, H), _BIG, dtype=jnp.float32)
    d2t = jax.lax.fori_loop(0, W, pass2, d20)

    dist_t = jnp.sqrt(d2t)

    preds = jax.nn.sigmoid(logits_ref[...])
    preds_t = jnp.swapaxes(preds, 1, 2)

    num1 = jnp.sum(preds_t * dist_t, axis=(1, 2))
    den1 = jnp.sum(preds, axis=(1, 2))
    num2 = jnp.sum((1.0 - preds) * maskf, axis=(1, 2))
    den2 = jnp.sum(maskf, axis=(1, 2))

    loss = num1 / den1 + num2 / den2
    out_ref[...] = jnp.reshape(jnp.sum(loss) / B, (1, 1))


@functools.partial(jax.jit, static_argnames=())
def _run(logits, targets):
    B, H, W = logits.shape
    out = pl.pallas_call(
        _hausdorff_body,
        out_shape=jax.ShapeDtypeStruct((1, 1), jnp.float32),
        scratch_shapes=[pltpu.VMEM((B, W, H), jnp.float32)],
    )(logits, targets.astype(jnp.int32))
    return out[0, 0]


def kernel(logits, targets):
    return _run(logits, targets)
